# baseline (speedup 1.0000x reference)
import functools

import jax
import jax.numpy as jnp
from jax import lax
from jax.experimental import pallas as pl
from jax.experimental.pallas import tpu as pltpu
from jax.experimental.pallas import tpu_sc as plsc

D = 64
C = 64
NW = 32
NBUF = 4


def _make_kernel(n):
    per_w = n // NW
    n_chunks = per_w // C
    mesh = plsc.VectorSubcoreMesh(core_axis_name="c", subcore_axis_name="s")

    @functools.partial(
        pl.kernel,
        mesh=mesh,
        out_type=jax.ShapeDtypeStruct((n, D), jnp.float32),
        compiler_params=pltpu.CompilerParams(use_tc_tiling_on_sc=False),
        scratch_types=[
            pltpu.VMEM((n_chunks, C), jnp.int32),
            pltpu.VMEM((n_chunks, C), jnp.int32),
            pltpu.VMEM((n_chunks, C), jnp.int32),
            [pltpu.VMEM((C, D), jnp.float32)] * NBUF,
            [pltpu.VMEM((C, D), jnp.float32)] * NBUF,
            [pltpu.VMEM((C, D), jnp.float32)] * NBUF,
            [pltpu.VMEM((C, D), jnp.float32)] * NBUF,
            [pltpu.SemaphoreType.DMA] * NBUF,
            [pltpu.SemaphoreType.DMA] * NBUF,
        ],
    )
    def k(h_idx_hbm, r_idx_hbm, t_idx_hbm, ent_hbm, rel_hbm, out_hbm,
          hi_v, ri_v, ti_v, h_bufs, r_bufs, t_bufs, o_bufs, g_sems, o_sems):
        wid = lax.axis_index("s") * 2 + lax.axis_index("c")
        base_w = wid * per_w

        pltpu.sync_copy(h_idx_hbm.at[wid], hi_v)
        pltpu.sync_copy(r_idx_hbm.at[wid], ri_v)
        pltpu.sync_copy(t_idx_hbm.at[wid], ti_v)

        def issue_gathers(c, b):
            pltpu.async_copy(ent_hbm.at[hi_v.at[c]], h_bufs[b], g_sems[b])
            pltpu.async_copy(rel_hbm.at[ri_v.at[c]], r_bufs[b], g_sems[b])
            pltpu.async_copy(ent_hbm.at[ti_v.at[c]], t_bufs[b], g_sems[b])

        def wait_gathers(b):
            pltpu.make_async_copy(ent_hbm.at[pl.ds(0, C)], h_bufs[b], g_sems[b]).wait()
            pltpu.make_async_copy(ent_hbm.at[pl.ds(0, C)], r_bufs[b], g_sems[b]).wait()
            pltpu.make_async_copy(ent_hbm.at[pl.ds(0, C)], t_bufs[b], g_sems[b]).wait()

        def wait_out(b):
            pltpu.make_async_copy(
                o_bufs[b], out_hbm.at[pl.ds(base_w, C)], o_sems[b]).wait()

        for b in range(NBUF):
            issue_gathers(b, b)

        def step(c, b):
            wait_gathers(b)

            @pl.when(c >= NBUF)
            def _():
                wait_out(b)

            h_v, r_v, t_v, o_v = h_bufs[b], r_bufs[b], t_bufs[b], o_bufs[b]

            def mul_row(i, carry):
                for kk in range(D // 16):
                    sl = pl.ds(kk * 16, 16)
                    o_v[i, sl] = h_v[i, sl] * r_v[i, sl] * t_v[i, sl]
                return carry

            lax.fori_loop(0, C, mul_row, 0, unroll=8)

            @pl.when(c + NBUF < n_chunks)
            def _():
                issue_gathers(c + NBUF, b)

            pltpu.async_copy(o_v, out_hbm.at[pl.ds(base_w + c * C, C)], o_sems[b])

        def outer(i, carry):
            for b in range(NBUF):
                step(i * NBUF + b, b)
            return carry

        lax.fori_loop(0, n_chunks // NBUF, outer, 0)

        for b in range(NBUF):
            wait_out(b)

    return k


def kernel(preds, subjs, objs, ent_emb, rel_emb):
    leading = preds.shape
    n = preds.size
    per_w = n // NW
    n_chunks = per_w // C
    h_flat = subjs.reshape(NW, n_chunks, C)
    r_flat = preds.reshape(NW, n_chunks, C)
    t_flat = objs.reshape(NW, n_chunks, C)
    out = _make_kernel(n)(h_flat, r_flat, t_flat, ent_emb, rel_emb)
    return out.reshape(*leading, D)


# speedup vs baseline: 1.0209x; 1.0209x over previous
import functools

import jax
import jax.numpy as jnp
from jax import lax
from jax.experimental import pallas as pl
from jax.experimental.pallas import tpu as pltpu
from jax.experimental.pallas import tpu_sc as plsc

D = 64
C = 128
NW = 32
NBUF = 2


def _make_kernel(n):
    per_w = n // NW
    n_chunks = per_w // C
    mesh = plsc.VectorSubcoreMesh(core_axis_name="c", subcore_axis_name="s")

    @functools.partial(
        pl.kernel,
        mesh=mesh,
        out_type=jax.ShapeDtypeStruct((n, D), jnp.float32),
        compiler_params=pltpu.CompilerParams(use_tc_tiling_on_sc=False),
        scratch_types=[
            pltpu.VMEM((n_chunks, C), jnp.int32),
            pltpu.VMEM((n_chunks, C), jnp.int32),
            pltpu.VMEM((n_chunks, C), jnp.int32),
            [pltpu.VMEM((C, D), jnp.float32)] * NBUF,
            [pltpu.VMEM((C, D), jnp.float32)] * NBUF,
            [pltpu.VMEM((C, D), jnp.float32)] * NBUF,
            [pltpu.VMEM((C, D), jnp.float32)] * NBUF,
            [pltpu.SemaphoreType.DMA] * NBUF,
            [pltpu.SemaphoreType.DMA] * NBUF,
        ],
    )
    def k(h_idx_hbm, r_idx_hbm, t_idx_hbm, ent_hbm, rel_hbm, out_hbm,
          hi_v, ri_v, ti_v, h_bufs, r_bufs, t_bufs, o_bufs, g_sems, o_sems):
        wid = lax.axis_index("s") * 2 + lax.axis_index("c")
        base_w = wid * per_w

        pltpu.sync_copy(h_idx_hbm.at[wid], hi_v)
        pltpu.sync_copy(r_idx_hbm.at[wid], ri_v)
        pltpu.sync_copy(t_idx_hbm.at[wid], ti_v)

        def issue_gathers(c, b):
            pltpu.async_copy(ent_hbm.at[hi_v.at[c]], h_bufs[b], g_sems[b])
            pltpu.async_copy(rel_hbm.at[ri_v.at[c]], r_bufs[b], g_sems[b])
            pltpu.async_copy(ent_hbm.at[ti_v.at[c]], t_bufs[b], g_sems[b])

        def wait_gathers(b):
            pltpu.make_async_copy(ent_hbm.at[pl.ds(0, C)], h_bufs[b], g_sems[b]).wait()
            pltpu.make_async_copy(ent_hbm.at[pl.ds(0, C)], r_bufs[b], g_sems[b]).wait()
            pltpu.make_async_copy(ent_hbm.at[pl.ds(0, C)], t_bufs[b], g_sems[b]).wait()

        def wait_out(b):
            pltpu.make_async_copy(
                o_bufs[b], out_hbm.at[pl.ds(base_w, C)], o_sems[b]).wait()

        for b in range(NBUF):
            issue_gathers(b, b)

        def step(c, b):
            wait_gathers(b)

            @pl.when(c >= NBUF)
            def _():
                wait_out(b)

            h_v, r_v, t_v, o_v = h_bufs[b], r_bufs[b], t_bufs[b], o_bufs[b]

            def mul_row(i, carry):
                for kk in range(D // 16):
                    sl = pl.ds(kk * 16, 16)
                    o_v[i, sl] = h_v[i, sl] * r_v[i, sl] * t_v[i, sl]
                return carry

            lax.fori_loop(0, C, mul_row, 0, unroll=8)

            @pl.when(c + NBUF < n_chunks)
            def _():
                issue_gathers(c + NBUF, b)

            pltpu.async_copy(o_v, out_hbm.at[pl.ds(base_w + c * C, C)], o_sems[b])

        def outer(i, carry):
            for b in range(NBUF):
                step(i * NBUF + b, b)
            return carry

        lax.fori_loop(0, n_chunks // NBUF, outer, 0)

        for b in range(NBUF):
            wait_out(b)

    return k


def _run_slab(preds, subjs, objs, ent_emb, rel_emb):
    b, f = preds.shape
    n = b * f
    n_chunks = n // NW // C
    h_flat = subjs.reshape(NW, n_chunks, C)
    r_flat = preds.reshape(NW, n_chunks, C)
    t_flat = objs.reshape(NW, n_chunks, C)
    out = _make_kernel(n)(h_flat, r_flat, t_flat, ent_emb, rel_emb)
    return out.reshape(b, f, D)


def kernel(preds, subjs, objs, ent_emb, rel_emb):
    f_half = preds.shape[1] // 2
    slabs = []
    for sl in (slice(0, f_half), slice(f_half, None)):
        slabs.append(_run_slab(preds[:, sl], subjs[:, sl], objs[:, sl],
                               ent_emb, rel_emb))
    return jnp.concatenate(slabs, axis=1)


# speedup vs baseline: 1.0220x; 1.0011x over previous
import functools

import jax
import jax.numpy as jnp
from jax import lax
from jax.experimental import pallas as pl
from jax.experimental.pallas import tpu as pltpu
from jax.experimental.pallas import tpu_sc as plsc

D = 64
C = 128
NW = 32
NBUF = 2


def _make_kernel(n):
    per_w = n // NW
    n_chunks = per_w // C
    mesh = plsc.VectorSubcoreMesh(core_axis_name="c", subcore_axis_name="s")

    @functools.partial(
        pl.kernel,
        mesh=mesh,
        out_type=jax.ShapeDtypeStruct((n, D), jnp.float32),
        compiler_params=pltpu.CompilerParams(use_tc_tiling_on_sc=False),
        scratch_types=[
            pltpu.VMEM((n_chunks, C), jnp.int32),
            pltpu.VMEM((n_chunks, C), jnp.int32),
            pltpu.VMEM((n_chunks, C), jnp.int32),
            [pltpu.VMEM((C, D), jnp.float32)] * NBUF,
            [pltpu.VMEM((C, D), jnp.float32)] * NBUF,
            [pltpu.VMEM((C, D), jnp.float32)] * NBUF,
            [pltpu.VMEM((C, D), jnp.float32)] * NBUF,
            [pltpu.SemaphoreType.DMA] * NBUF,
            [pltpu.SemaphoreType.DMA] * NBUF,
        ],
    )
    def k(h_idx_hbm, r_idx_hbm, t_idx_hbm, ent_hbm, rel_hbm, out_hbm,
          hi_v, ri_v, ti_v, h_bufs, r_bufs, t_bufs, o_bufs, g_sems, o_sems):
        wid = lax.axis_index("s") * 2 + lax.axis_index("c")
        base_w = wid * per_w

        pltpu.sync_copy(h_idx_hbm.at[wid], hi_v)
        pltpu.sync_copy(r_idx_hbm.at[wid], ri_v)
        pltpu.sync_copy(t_idx_hbm.at[wid], ti_v)

        def issue_gathers(c, b):
            pltpu.async_copy(ent_hbm.at[hi_v.at[c]], h_bufs[b], g_sems[b])
            pltpu.async_copy(rel_hbm.at[ri_v.at[c]], r_bufs[b], g_sems[b])
            pltpu.async_copy(ent_hbm.at[ti_v.at[c]], t_bufs[b], g_sems[b])

        def wait_gathers(b):
            pltpu.make_async_copy(ent_hbm.at[pl.ds(0, C)], h_bufs[b], g_sems[b]).wait()
            pltpu.make_async_copy(ent_hbm.at[pl.ds(0, C)], r_bufs[b], g_sems[b]).wait()
            pltpu.make_async_copy(ent_hbm.at[pl.ds(0, C)], t_bufs[b], g_sems[b]).wait()

        def wait_out(b):
            pltpu.make_async_copy(
                o_bufs[b], out_hbm.at[pl.ds(base_w, C)], o_sems[b]).wait()

        for b in range(NBUF):
            issue_gathers(b, b)

        def step(c, b):
            wait_gathers(b)

            @pl.when(c >= NBUF)
            def _():
                wait_out(b)

            h_v, r_v, t_v, o_v = h_bufs[b], r_bufs[b], t_bufs[b], o_bufs[b]

            def mul_row(i, carry):
                for kk in range(D // 16):
                    sl = pl.ds(kk * 16, 16)
                    o_v[i, sl] = h_v[i, sl] * r_v[i, sl] * t_v[i, sl]
                return carry

            lax.fori_loop(0, C, mul_row, 0, unroll=8)

            @pl.when(c + NBUF < n_chunks)
            def _():
                issue_gathers(c + NBUF, b)

            pltpu.async_copy(o_v, out_hbm.at[pl.ds(base_w + c * C, C)], o_sems[b])

        def outer(i, carry):
            for b in range(NBUF):
                step(i * NBUF + b, b)
            return carry

        lax.fori_loop(0, n_chunks // NBUF, outer, 0)

        for b in range(NBUF):
            wait_out(b)

    return k


def _run_slab(preds, subjs, objs, ent_tbl, rel_emb):
    b, f = preds.shape
    n = b * f
    n_chunks = n // NW // C
    h_flat = subjs.reshape(NW, n_chunks, C)
    r_flat = preds.reshape(NW, n_chunks, C)
    t_flat = objs.reshape(NW, n_chunks, C)
    out = _make_kernel(n)(h_flat, r_flat, t_flat, ent_tbl, rel_emb)
    return out.reshape(b, f, D)


def kernel(preds, subjs, objs, ent_emb, rel_emb):
    f_half = preds.shape[1] // 2
    slabs = []
    for sl in (slice(0, f_half), slice(f_half, None)):
        slabs.append(_run_slab(preds[:, sl], subjs[:, sl], objs[:, sl],
                               ent_emb, rel_emb))
    return jnp.concatenate(slabs, axis=1)
